# baseline (speedup 1.0000x reference)
import functools

import jax
import jax.numpy as jnp
from jax import lax
from jax.experimental import pallas as pl
from jax.experimental.pallas import tpu as pltpu
from jax.experimental.pallas import tpu_sc as plsc

M = 524288
D = 128
A = 131072
N = 65536
TOT = A + N

NC = 2
NS = 16
NW = NC * NS
L = 16

RPW = M // NW
ICH = 8192
NICH = TOT // ICH
G = 128
ARENA = RPW + 2 * G + L


def _fori(lo, hi, body, init, unroll=None):
    if unroll is None or unroll == 1:
        return lax.fori_loop(jnp.int32(lo), jnp.int32(hi), body, init)
    assert lo == 0 and isinstance(hi, int) and hi % unroll == 0

    def blk(k, carry):
        for u in range(unroll):
            carry = body(k * unroll + u, carry)
        return carry

    return lax.fori_loop(jnp.int32(0), jnp.int32(hi // unroll), blk, init)


def _popcnt(m):
    return jnp.max(plsc.all_reduce_population_count(m))


def _sc_body(mem_hbm, obs_hbm, new_hbm, slots_hbm, out_hbm,
             winner, idxbuf, dst_arena, src_arena, stage,
             isem, gsem, ssem):
    wid = lax.axis_index("s") * NC + lax.axis_index("c")
    base = wid * RPW
    iota = lax.iota(jnp.int32, L)
    zid = jnp.zeros((L,), jnp.int32)
    neg1 = jnp.full((L,), -1, jnp.int32)

    def init_body(k, carry):
        winner[pl.ds(k * L, L)] = neg1
        return carry

    with jax.named_scope("ph_init"):
        _fori(0, RPW // L, init_body, 0, unroll=8)

    def istart(c, b):
        bi = jnp.int32(b)
        return pltpu.async_copy(slots_hbm.at[pl.ds(c * ICH, ICH)],
                                idxbuf.at[bi], isem.at[bi])

    def iwait(c, b):
        bi = jnp.int32(b)
        pltpu.make_async_copy(slots_hbm.at[pl.ds(c * ICH, ICH)],
                              idxbuf.at[bi], isem.at[bi]).wait()

    istart(jnp.int32(0), 0)

    def process_chunk(c, b):
        c0 = c * ICH

        def vec_body(v, inner):
            s = idxbuf[jnp.int32(b), pl.ds(v * L, L)]
            local = s - base
            m = (local >= 0) & (local < RPW)
            localc = local & (RPW - 1)
            ivec = c0 + v * L + iota

            def wcond(nleft):
                return nleft > 0

            def wbody(_):
                cur = plsc.load_gather(winner, [localc], mask=m)
                upd = m & (ivec > cur)
                plsc.store_scatter(winner, [localc], ivec, mask=upd)
                return _popcnt(upd)

            lax.while_loop(wcond, wbody, _popcnt(m))
            return inner

        _fori(0, ICH // L, vec_body, 0, unroll=4)

    def pair_body(k, carry):
        c = 2 * k
        iwait(c, 0)
        istart(c + 1, 1)
        process_chunk(c, 0)
        iwait(c + 1, 1)

        @pl.when(k < NICH // 2 - 1)
        def _():
            istart(c + 2, 0)

        process_chunk(c + 1, 1)
        return carry

    with jax.named_scope("ph_winner"):
        _fori(0, NICH // 2, pair_body, 0)

    NCH = RPW // G

    def cin(j, b):
        return (mem_hbm.at[pl.ds(base + j * G, G)], stage.at[jnp.int32(b)],
                gsem.at[jnp.int32(b)])

    def cout(j, b):
        return (stage.at[jnp.int32(b)], out_hbm.at[pl.ds(base + j * G, G)],
                ssem.at[jnp.int32(b)])

    pltpu.async_copy(*cin(jnp.int32(0), 0))
    pltpu.async_copy(*cin(jnp.int32(1), 1))

    def cpair(k, carry):
        for b in (0, 1):
            j = 2 * k + b
            pltpu.make_async_copy(*cin(j, b)).wait()
            pltpu.async_copy(*cout(j, b))

            @pl.when(j + 2 < NCH)
            def _():
                pltpu.make_async_copy(*cout(j, b)).wait()
                pltpu.async_copy(*cin(j + 2, b))

        return carry

    with jax.named_scope("ph_copy"):
        _fori(0, NCH // 2, cpair, 0)
        pltpu.make_async_copy(*cout(jnp.int32(NCH - 2), 0)).wait()
        pltpu.make_async_copy(*cout(jnp.int32(NCH - 1), 1)).wait()

    def compact(off0, lo_i, hi_i, sub_i):
        def body(v, cnt):
            w = winner[pl.ds(v * L, L)]
            m = (w >= lo_i) & (w < hi_i)
            dst = base + v * L + iota
            plsc.store_compressed(dst_arena.at[pl.ds(off0 + cnt, L)], dst,
                                  mask=m)
            plsc.store_compressed(src_arena.at[pl.ds(off0 + cnt, L)],
                                  w - sub_i, mask=m)
            return cnt + _popcnt(m)

        return _fori(0, RPW // L, body, jnp.int32(0))

    def pad(off0, cnt):
        bd = jnp.take(dst_arena[pl.ds(off0, L)], zid)
        bs = jnp.take(src_arena[pl.ds(off0, L)], zid)
        for t in range(G // L):
            dst_arena[pl.ds(off0 + cnt + t * L, L)] = bd
            src_arena[pl.ds(off0 + cnt + t * L, L)] = bs

    with jax.named_scope("ph_compact"):
        cnt_obs = compact(jnp.int32(0), jnp.int32(0), jnp.int32(A), jnp.int32(0))
        pad(jnp.int32(0), cnt_obs)
        q_obs = (cnt_obs + (G - 1)) // G
        off_new = q_obs * G
        cnt_new = compact(off_new, jnp.int32(A), jnp.int32(TOT),
                          jnp.int32(A))
        pad(off_new, cnt_new)
        q_new = (cnt_new + (G - 1)) // G


    def gs_loop(feat_hbm, off0, q):
        def gstart(j, b):
            pltpu.async_copy(
                feat_hbm.at[src_arena.at[pl.ds(off0 + j * G, G)]],
                stage.at[b], gsem.at[b])

        def gwait(j, b):
            pltpu.make_async_copy(
                feat_hbm.at[src_arena.at[pl.ds(off0 + j * G, G)]],
                stage.at[b], gsem.at[b]).wait()

        def sstart(j, b):
            pltpu.async_copy(
                stage.at[b],
                out_hbm.at[dst_arena.at[pl.ds(off0 + j * G, G)]],
                ssem.at[b])

        def swait(j, b):
            pltpu.make_async_copy(
                stage.at[b],
                out_hbm.at[dst_arena.at[pl.ds(off0 + j * G, G)]],
                ssem.at[b]).wait()

        @pl.when(q > 0)
        def _():
            gstart(jnp.int32(0), jnp.int32(0))

        def body(j, carry):
            b = j % 2
            gwait(j, b)
            sstart(j, b)

            @pl.when(j + 1 < q)
            def _():
                @pl.when(j >= 1)
                def _():
                    swait(j - 1, 1 - b)

                gstart(j + 1, 1 - b)

            return carry

        _fori(0, q, body, 0)

        @pl.when(q > 1)
        def _():
            swait(q - 2, (q - 2) % 2)

        @pl.when(q > 0)
        def _():
            swait(q - 1, (q - 1) % 2)

    with jax.named_scope("ph_gs"):
        gs_loop(obs_hbm, jnp.int32(0), q_obs)
        gs_loop(new_hbm, off_new, q_new)


_sc_scatter = functools.partial(
    pl.kernel,
    out_type=jax.ShapeDtypeStruct((M, D), jnp.float32),
    mesh=plsc.VectorSubcoreMesh(core_axis_name="c", subcore_axis_name="s"),
    scratch_types=[
        pltpu.VMEM((RPW,), jnp.int32),
        pltpu.VMEM((2, ICH), jnp.int32),
        pltpu.VMEM((ARENA,), jnp.int32),
        pltpu.VMEM((ARENA,), jnp.int32),
        pltpu.VMEM((2, G, D), jnp.float32),
        pltpu.SemaphoreType.DMA((2,)),
        pltpu.SemaphoreType.DMA((2,)),
        pltpu.SemaphoreType.DMA((2,)),
    ],
    compiler_params=pltpu.CompilerParams(needs_layout_passes=False),
)(_sc_body)


def _ids_body(act_ref, out_ref):
    act = act_ref[...]
    mx = jnp.max(act)
    out_ref[0:A // D, :] = act
    r = lax.broadcasted_iota(jnp.int32, (N // D, D), 0)
    c = lax.broadcasted_iota(jnp.int32, (N // D, D), 1)
    out_ref[A // D:(A + N) // D, :] = mx + 1 + r * D + c


_ids_kernel = pl.pallas_call(
    _ids_body,
    out_shape=jax.ShapeDtypeStruct(((A + N) // D, D), jnp.int32),
)


def kernel(mem, obs_feat, new_feat, obs_slots, new_slots, active_ids,
           active_det_idx):
    slots = jnp.concatenate([obs_slots, new_slots]).astype(jnp.int32)
    new_mem = _sc_scatter(mem, obs_feat, new_feat, slots)
    act2d = active_ids.astype(jnp.int32).reshape(A // D, D)
    ids = _ids_kernel(act2d).reshape(-1).astype(active_ids.dtype)
    return (new_mem, ids)


# speedup vs baseline: 1.6750x; 1.6750x over previous
import functools

import jax
import jax.numpy as jnp
from jax import lax
from jax.experimental import pallas as pl
from jax.experimental.pallas import tpu as pltpu
from jax.experimental.pallas import tpu_sc as plsc

M = 524288
D = 128
A = 131072
N = 65536
TOT = A + N

NC = 2
NS = 16
NW = NC * NS
L = 16

RPW = M // NW
ICH = 8192
NICH = TOT // ICH
G = 128
ARENA = RPW + 2 * G + L


def _fori(lo, hi, body, init, unroll=None):
    if unroll is None or unroll == 1:
        return lax.fori_loop(jnp.int32(lo), jnp.int32(hi), body, init)
    assert lo == 0 and isinstance(hi, int) and hi % unroll == 0

    def blk(k, carry):
        for u in range(unroll):
            carry = body(k * unroll + u, carry)
        return carry

    return lax.fori_loop(jnp.int32(0), jnp.int32(hi // unroll), blk, init)


def _popcnt(m):
    return jnp.max(plsc.all_reduce_population_count(m))


def _sc_body(mem_hbm, obs_hbm, new_hbm, slots_hbm, out_hbm,
             winner, idxbuf, dst_arena, src_arena, stage,
             isem, gsem, ssem):
    wid = lax.axis_index("s") * NC + lax.axis_index("c")
    base = wid * RPW
    iota = lax.iota(jnp.int32, L)
    zid = jnp.zeros((L,), jnp.int32)
    neg1 = jnp.full((L,), -1, jnp.int32)

    def init_body(k, carry):
        winner[pl.ds(k * L, L)] = neg1
        return carry

    with jax.named_scope("ph_init"):
        _fori(0, RPW // L, init_body, 0, unroll=8)

    def istart(c, b):
        bi = jnp.int32(b)
        return pltpu.async_copy(slots_hbm.at[pl.ds(c * ICH, ICH)],
                                idxbuf.at[bi], isem.at[bi])

    def iwait(c, b):
        bi = jnp.int32(b)
        pltpu.make_async_copy(slots_hbm.at[pl.ds(c * ICH, ICH)],
                              idxbuf.at[bi], isem.at[bi]).wait()

    istart(jnp.int32(0), 0)

    def process_chunk(c, b):
        c0 = c * ICH

        def one_pass(_):
            def vec_body(v, accv):
                s = idxbuf[jnp.int32(b), pl.ds(v * L, L)]
                local = s - base
                inr = plsc.bitcast(local, jnp.uint32) < jnp.uint32(RPW)
                localc = local & (RPW - 1)
                ivec = c0 + v * L + iota
                cur = plsc.load_gather(winner, [localc], mask=inr)
                upd = inr & (ivec > cur)
                plsc.store_scatter(winner, [localc], ivec, mask=upd)
                cur2 = plsc.load_gather(winner, [localc], mask=upd)
                lost = upd & (cur2 < ivec)
                return accv | lost.astype(jnp.int32)

            accv = _fori(0, ICH // L, vec_body,
                         jnp.zeros((L,), jnp.int32), unroll=8)
            return jnp.max(accv)

        lax.while_loop(lambda t: t > 0, one_pass, one_pass(jnp.int32(1)))

    def pair_body(k, carry):
        c = 2 * k
        iwait(c, 0)
        istart(c + 1, 1)
        process_chunk(c, 0)
        iwait(c + 1, 1)

        @pl.when(k < NICH // 2 - 1)
        def _():
            istart(c + 2, 0)

        process_chunk(c + 1, 1)
        return carry

    with jax.named_scope("ph_winner"):
        _fori(0, NICH // 2, pair_body, 0)

    NCH = RPW // G

    def cin(j, b):
        return (mem_hbm.at[pl.ds(base + j * G, G)], stage.at[jnp.int32(b)],
                gsem.at[jnp.int32(b)])

    def cout(j, b):
        return (stage.at[jnp.int32(b)], out_hbm.at[pl.ds(base + j * G, G)],
                ssem.at[jnp.int32(b)])

    pltpu.async_copy(*cin(jnp.int32(0), 0))
    pltpu.async_copy(*cin(jnp.int32(1), 1))

    def cpair(k, carry):
        for b in (0, 1):
            j = 2 * k + b
            pltpu.make_async_copy(*cin(j, b)).wait()
            pltpu.async_copy(*cout(j, b))

            @pl.when(j + 2 < NCH)
            def _():
                pltpu.make_async_copy(*cout(j, b)).wait()
                pltpu.async_copy(*cin(j + 2, b))

        return carry

    with jax.named_scope("ph_copy"):
        _fori(0, NCH // 2, cpair, 0)
        pltpu.make_async_copy(*cout(jnp.int32(NCH - 2), 0)).wait()
        pltpu.make_async_copy(*cout(jnp.int32(NCH - 1), 1)).wait()

    def compact(off0, lo_i, hi_i, sub_i):
        def body(v, cnt):
            w = winner[pl.ds(v * L, L)]
            m = (w >= lo_i) & (w < hi_i)
            dst = base + v * L + iota
            plsc.store_compressed(dst_arena.at[pl.ds(off0 + cnt, L)], dst,
                                  mask=m)
            plsc.store_compressed(src_arena.at[pl.ds(off0 + cnt, L)],
                                  w - sub_i, mask=m)
            return cnt + _popcnt(m)

        return _fori(0, RPW // L, body, jnp.int32(0))

    def pad(off0, cnt):
        bd = jnp.take(dst_arena[pl.ds(off0, L)], zid)
        bs = jnp.take(src_arena[pl.ds(off0, L)], zid)
        for t in range(G // L):
            dst_arena[pl.ds(off0 + cnt + t * L, L)] = bd
            src_arena[pl.ds(off0 + cnt + t * L, L)] = bs

    with jax.named_scope("ph_compact"):
        cnt_obs = compact(jnp.int32(0), jnp.int32(0), jnp.int32(A), jnp.int32(0))
        pad(jnp.int32(0), cnt_obs)
        q_obs = (cnt_obs + (G - 1)) // G
        off_new = q_obs * G
        cnt_new = compact(off_new, jnp.int32(A), jnp.int32(TOT),
                          jnp.int32(A))
        pad(off_new, cnt_new)
        q_new = (cnt_new + (G - 1)) // G


    def gs_loop(feat_hbm, off0, q):
        def gstart(j, b):
            pltpu.async_copy(
                feat_hbm.at[src_arena.at[pl.ds(off0 + j * G, G)]],
                stage.at[b], gsem.at[b])

        def gwait(j, b):
            pltpu.make_async_copy(
                feat_hbm.at[src_arena.at[pl.ds(off0 + j * G, G)]],
                stage.at[b], gsem.at[b]).wait()

        def sstart(j, b):
            pltpu.async_copy(
                stage.at[b],
                out_hbm.at[dst_arena.at[pl.ds(off0 + j * G, G)]],
                ssem.at[b])

        def swait(j, b):
            pltpu.make_async_copy(
                stage.at[b],
                out_hbm.at[dst_arena.at[pl.ds(off0 + j * G, G)]],
                ssem.at[b]).wait()

        @pl.when(q > 0)
        def _():
            gstart(jnp.int32(0), jnp.int32(0))

        def body(j, carry):
            b = j % 2
            gwait(j, b)
            sstart(j, b)

            @pl.when(j + 1 < q)
            def _():
                @pl.when(j >= 1)
                def _():
                    swait(j - 1, 1 - b)

                gstart(j + 1, 1 - b)

            return carry

        _fori(0, q, body, 0)

        @pl.when(q > 1)
        def _():
            swait(q - 2, (q - 2) % 2)

        @pl.when(q > 0)
        def _():
            swait(q - 1, (q - 1) % 2)

    with jax.named_scope("ph_gs"):
        gs_loop(obs_hbm, jnp.int32(0), q_obs)
        gs_loop(new_hbm, off_new, q_new)


_sc_scatter = functools.partial(
    pl.kernel,
    out_type=jax.ShapeDtypeStruct((M, D), jnp.float32),
    mesh=plsc.VectorSubcoreMesh(core_axis_name="c", subcore_axis_name="s"),
    scratch_types=[
        pltpu.VMEM((RPW,), jnp.int32),
        pltpu.VMEM((2, ICH), jnp.int32),
        pltpu.VMEM((ARENA,), jnp.int32),
        pltpu.VMEM((ARENA,), jnp.int32),
        pltpu.VMEM((2, G, D), jnp.float32),
        pltpu.SemaphoreType.DMA((2,)),
        pltpu.SemaphoreType.DMA((2,)),
        pltpu.SemaphoreType.DMA((2,)),
    ],
    compiler_params=pltpu.CompilerParams(needs_layout_passes=False),
)(_sc_body)


def _ids_body(act_ref, out_ref):
    act = act_ref[...]
    mx = jnp.max(act)
    out_ref[0:A // D, :] = act
    r = lax.broadcasted_iota(jnp.int32, (N // D, D), 0)
    c = lax.broadcasted_iota(jnp.int32, (N // D, D), 1)
    out_ref[A // D:(A + N) // D, :] = mx + 1 + r * D + c


_ids_kernel = pl.pallas_call(
    _ids_body,
    out_shape=jax.ShapeDtypeStruct(((A + N) // D, D), jnp.int32),
)


def kernel(mem, obs_feat, new_feat, obs_slots, new_slots, active_ids,
           active_det_idx):
    slots = jnp.concatenate([obs_slots, new_slots]).astype(jnp.int32)
    new_mem = _sc_scatter(mem, obs_feat, new_feat, slots)
    act2d = active_ids.astype(jnp.int32).reshape(A // D, D)
    ids = _ids_kernel(act2d).reshape(-1).astype(active_ids.dtype)
    return (new_mem, ids)


# speedup vs baseline: 1.8346x; 1.0953x over previous
import functools

import jax
import jax.numpy as jnp
from jax import lax
from jax.experimental import pallas as pl
from jax.experimental.pallas import tpu as pltpu
from jax.experimental.pallas import tpu_sc as plsc

M = 524288
D = 128
A = 131072
N = 65536
TOT = A + N

NC = 2
NS = 16
NW = NC * NS
L = 16

RPW = M // NW
ICH = 8192
NICH = TOT // ICH
G = 128
ARENA = RPW + 2 * G + L


def _fori(lo, hi, body, init, unroll=None):
    if unroll is None or unroll == 1:
        return lax.fori_loop(jnp.int32(lo), jnp.int32(hi), body, init)
    assert lo == 0 and isinstance(hi, int) and hi % unroll == 0

    def blk(k, carry):
        for u in range(unroll):
            carry = body(k * unroll + u, carry)
        return carry

    return lax.fori_loop(jnp.int32(0), jnp.int32(hi // unroll), blk, init)


def _popcnt(m):
    return jnp.max(plsc.all_reduce_population_count(m))


def _sc_body(mem_hbm, obs_hbm, new_hbm, slots_hbm, out_hbm,
             winner, idxbuf, dst_arena, src_arena, stage,
             isem, gsem, ssem):
    wid = lax.axis_index("s") * NC + lax.axis_index("c")
    base = wid * RPW
    iota = lax.iota(jnp.int32, L)
    zid = jnp.zeros((L,), jnp.int32)
    neg1 = jnp.full((L,), -1, jnp.int32)

    def init_body(k, carry):
        winner[pl.ds(k * L, L)] = neg1
        return carry

    with jax.named_scope("ph_init"):
        _fori(0, RPW // L, init_body, 0, unroll=8)

    def istart(c, b):
        bi = jnp.int32(b)
        return pltpu.async_copy(slots_hbm.at[pl.ds(c * ICH, ICH)],
                                idxbuf.at[bi], isem.at[bi])

    def iwait(c, b):
        bi = jnp.int32(b)
        pltpu.make_async_copy(slots_hbm.at[pl.ds(c * ICH, ICH)],
                              idxbuf.at[bi], isem.at[bi]).wait()

    istart(jnp.int32(0), 0)

    def process_chunk(c, b):
        c0 = c * ICH

        def one_pass(_):
            def vec_body(v, accv):
                s = idxbuf[jnp.int32(b), pl.ds(v * L, L)]
                local = s - base
                inr = plsc.bitcast(local, jnp.uint32) < jnp.uint32(RPW)
                localc = local & (RPW - 1)
                ivec = c0 + v * L + iota
                cur = plsc.load_gather(winner, [localc], mask=inr)
                upd = inr & (ivec > cur)
                plsc.store_scatter(winner, [localc], ivec, mask=upd)
                cur2 = plsc.load_gather(winner, [localc], mask=upd)
                lost = upd & (cur2 < ivec)
                return accv | lost.astype(jnp.int32)

            accv = _fori(0, ICH // L, vec_body,
                         jnp.zeros((L,), jnp.int32), unroll=8)
            return jnp.max(accv)

        lax.while_loop(lambda t: t > 0, one_pass, one_pass(jnp.int32(1)))

    NCH = RPW // G
    CSPP = (NCH + NICH // 2 - 1) // (NICH // 2)

    def cin(j, b):
        return (mem_hbm.at[pl.ds(base + j * G, G)], stage.at[jnp.int32(b)],
                gsem.at[jnp.int32(b)])

    def cout(j, b):
        return (stage.at[jnp.int32(b)], out_hbm.at[pl.ds(base + j * G, G)],
                ssem.at[jnp.int32(b)])

    def cstep(j):
        @pl.when(j < NCH)
        def _():
            b = j % 2
            pltpu.make_async_copy(*cin(j, b)).wait()
            pltpu.async_copy(*cout(j, b))

            @pl.when(j + 2 < NCH)
            def _():
                pltpu.make_async_copy(*cout(j, b)).wait()
                pltpu.async_copy(*cin(j + 2, b))

    pltpu.async_copy(*cin(jnp.int32(0), 0))
    pltpu.async_copy(*cin(jnp.int32(1), 1))

    def pair_body(k, carry):
        c = 2 * k
        iwait(c, 0)
        istart(c + 1, 1)
        process_chunk(c, 0)
        for t in range(CSPP // 2):
            cstep(k * CSPP + t)
        iwait(c + 1, 1)

        @pl.when(k < NICH // 2 - 1)
        def _():
            istart(c + 2, 0)

        process_chunk(c + 1, 1)
        for t in range(CSPP // 2, CSPP):
            cstep(k * CSPP + t)
        return carry

    with jax.named_scope("ph_winner"):
        _fori(0, NICH // 2, pair_body, 0)

    with jax.named_scope("ph_copy"):
        pltpu.make_async_copy(*cout(jnp.int32(NCH - 2), 0)).wait()
        pltpu.make_async_copy(*cout(jnp.int32(NCH - 1), 1)).wait()

    def compact(off0, lo_i, hi_i, sub_i):
        def body(v, cnt):
            w = winner[pl.ds(v * L, L)]
            m = (w >= lo_i) & (w < hi_i)
            dst = base + v * L + iota
            plsc.store_compressed(dst_arena.at[pl.ds(off0 + cnt, L)], dst,
                                  mask=m)
            plsc.store_compressed(src_arena.at[pl.ds(off0 + cnt, L)],
                                  w - sub_i, mask=m)
            return cnt + _popcnt(m)

        return _fori(0, RPW // L, body, jnp.int32(0))

    def pad(off0, cnt):
        bd = jnp.take(dst_arena[pl.ds(off0, L)], zid)
        bs = jnp.take(src_arena[pl.ds(off0, L)], zid)
        for t in range(G // L):
            dst_arena[pl.ds(off0 + cnt + t * L, L)] = bd
            src_arena[pl.ds(off0 + cnt + t * L, L)] = bs

    with jax.named_scope("ph_compact"):
        cnt_obs = compact(jnp.int32(0), jnp.int32(0), jnp.int32(A), jnp.int32(0))
        pad(jnp.int32(0), cnt_obs)
        q_obs = (cnt_obs + (G - 1)) // G
        off_new = q_obs * G
        cnt_new = compact(off_new, jnp.int32(A), jnp.int32(TOT),
                          jnp.int32(A))
        pad(off_new, cnt_new)
        q_new = (cnt_new + (G - 1)) // G


    def gs_loop(feat_hbm, off0, q):
        def gstart(j, b):
            pltpu.async_copy(
                feat_hbm.at[src_arena.at[pl.ds(off0 + j * G, G)]],
                stage.at[b], gsem.at[b])

        def gwait(j, b):
            pltpu.make_async_copy(
                feat_hbm.at[src_arena.at[pl.ds(off0 + j * G, G)]],
                stage.at[b], gsem.at[b]).wait()

        def sstart(j, b):
            pltpu.async_copy(
                stage.at[b],
                out_hbm.at[dst_arena.at[pl.ds(off0 + j * G, G)]],
                ssem.at[b])

        def swait(j, b):
            pltpu.make_async_copy(
                stage.at[b],
                out_hbm.at[dst_arena.at[pl.ds(off0 + j * G, G)]],
                ssem.at[b]).wait()

        @pl.when(q > 0)
        def _():
            gstart(jnp.int32(0), jnp.int32(0))

        def body(j, carry):
            b = j % 2
            gwait(j, b)
            sstart(j, b)

            @pl.when(j + 1 < q)
            def _():
                @pl.when(j >= 1)
                def _():
                    swait(j - 1, 1 - b)

                gstart(j + 1, 1 - b)

            return carry

        _fori(0, q, body, 0)

        @pl.when(q > 1)
        def _():
            swait(q - 2, (q - 2) % 2)

        @pl.when(q > 0)
        def _():
            swait(q - 1, (q - 1) % 2)

    with jax.named_scope("ph_gs"):
        gs_loop(obs_hbm, jnp.int32(0), q_obs)
        gs_loop(new_hbm, off_new, q_new)


_sc_scatter = functools.partial(
    pl.kernel,
    out_type=jax.ShapeDtypeStruct((M, D), jnp.float32),
    mesh=plsc.VectorSubcoreMesh(core_axis_name="c", subcore_axis_name="s"),
    scratch_types=[
        pltpu.VMEM((RPW,), jnp.int32),
        pltpu.VMEM((2, ICH), jnp.int32),
        pltpu.VMEM((ARENA,), jnp.int32),
        pltpu.VMEM((ARENA,), jnp.int32),
        pltpu.VMEM((2, G, D), jnp.float32),
        pltpu.SemaphoreType.DMA((2,)),
        pltpu.SemaphoreType.DMA((2,)),
        pltpu.SemaphoreType.DMA((2,)),
    ],
    compiler_params=pltpu.CompilerParams(needs_layout_passes=False),
)(_sc_body)


def _ids_body(act_ref, out_ref):
    act = act_ref[...]
    mx = jnp.max(act)
    out_ref[0:A // D, :] = act
    r = lax.broadcasted_iota(jnp.int32, (N // D, D), 0)
    c = lax.broadcasted_iota(jnp.int32, (N // D, D), 1)
    out_ref[A // D:(A + N) // D, :] = mx + 1 + r * D + c


_ids_kernel = pl.pallas_call(
    _ids_body,
    out_shape=jax.ShapeDtypeStruct(((A + N) // D, D), jnp.int32),
)


def kernel(mem, obs_feat, new_feat, obs_slots, new_slots, active_ids,
           active_det_idx):
    slots = jnp.concatenate([obs_slots, new_slots]).astype(jnp.int32)
    new_mem = _sc_scatter(mem, obs_feat, new_feat, slots)
    act2d = active_ids.astype(jnp.int32).reshape(A // D, D)
    ids = _ids_kernel(act2d).reshape(-1).astype(active_ids.dtype)
    return (new_mem, ids)


# speedup vs baseline: 2.1003x; 1.1448x over previous
import functools

import jax
import jax.numpy as jnp
from jax import lax
from jax.experimental import pallas as pl
from jax.experimental.pallas import tpu as pltpu
from jax.experimental.pallas import tpu_sc as plsc

M = 524288
D = 128
A = 131072
N = 65536
TOT = A + N

NC = 2
NS = 16
NW = NC * NS
L = 16

RPW = M // NW
ICH = 4096
NICH = TOT // ICH
G = 128
ARENA = RPW + 2 * G + L


def _fori(lo, hi, body, init, unroll=None):
    if unroll is None or unroll == 1:
        return lax.fori_loop(jnp.int32(lo), jnp.int32(hi), body, init)
    assert lo == 0 and isinstance(hi, int) and hi % unroll == 0

    def blk(k, carry):
        for u in range(unroll):
            carry = body(k * unroll + u, carry)
        return carry

    return lax.fori_loop(jnp.int32(0), jnp.int32(hi // unroll), blk, init)


def _popcnt(m):
    return jnp.max(plsc.all_reduce_population_count(m))


def _sc_body(mem_hbm, obs_hbm, new_hbm, slots_hbm, out_hbm,
             winner, idxbuf, dst_arena, src_arena, stage,
             isem, gsem, ssem):
    wid = lax.axis_index("s") * NC + lax.axis_index("c")
    base = wid * RPW
    iota = lax.iota(jnp.int32, L)
    zid = jnp.zeros((L,), jnp.int32)
    neg1 = jnp.full((L,), -1, jnp.int32)

    def init_body(k, carry):
        winner[pl.ds(k * L, L)] = neg1
        return carry

    with jax.named_scope("ph_init"):
        _fori(0, RPW // L, init_body, 0, unroll=8)

    def istart(c, b):
        bi = jnp.int32(b)
        return pltpu.async_copy(slots_hbm.at[pl.ds(c * ICH, ICH)],
                                idxbuf.at[bi], isem.at[bi])

    def iwait(c, b):
        bi = jnp.int32(b)
        pltpu.make_async_copy(slots_hbm.at[pl.ds(c * ICH, ICH)],
                              idxbuf.at[bi], isem.at[bi]).wait()

    istart(jnp.int32(0), 0)

    def process_chunk(c, b):
        c0 = c * ICH

        def vec_blind(v, accv):
            s = idxbuf[jnp.int32(b), pl.ds(v * L, L)]
            local = s - base
            inr = plsc.bitcast(local, jnp.uint32) < jnp.uint32(RPW)
            localc = local & (RPW - 1)
            ivec = c0 + v * L + iota
            plsc.store_scatter(winner, [localc], ivec, mask=inr)
            cur2 = plsc.load_gather(winner, [localc], mask=inr)
            lost = inr & (cur2 < ivec)
            return accv | lost.astype(jnp.int32)

        def vec_rmw(v, accv):
            s = idxbuf[jnp.int32(b), pl.ds(v * L, L)]
            local = s - base
            inr = plsc.bitcast(local, jnp.uint32) < jnp.uint32(RPW)
            localc = local & (RPW - 1)
            ivec = c0 + v * L + iota
            cur = plsc.load_gather(winner, [localc], mask=inr)
            upd = inr & (ivec > cur)
            plsc.store_scatter(winner, [localc], ivec, mask=upd)
            cur2 = plsc.load_gather(winner, [localc], mask=upd)
            lost = upd & (cur2 < ivec)
            return accv | lost.astype(jnp.int32)

        def run(vec_body):
            accv = _fori(0, ICH // L, vec_body,
                         jnp.zeros((L,), jnp.int32), unroll=8)
            return jnp.max(accv)

        lax.while_loop(lambda t: t > 0, lambda _: run(vec_rmw),
                       run(vec_blind))

    NCH = RPW // G
    CSPP = (NCH + NICH // 2 - 1) // (NICH // 2)

    def cin(j, b):
        return (mem_hbm.at[pl.ds(base + j * G, G)], stage.at[jnp.int32(b)],
                gsem.at[jnp.int32(b)])

    def cout(j, b):
        return (stage.at[jnp.int32(b)], out_hbm.at[pl.ds(base + j * G, G)],
                ssem.at[jnp.int32(b)])

    def cstep(j):
        @pl.when(j < NCH)
        def _():
            b = j % 4
            pltpu.make_async_copy(*cin(j, b)).wait()
            pltpu.async_copy(*cout(j, b))

            @pl.when(j + 2 < NCH)
            def _():
                b2 = (j + 2) % 4

                @pl.when(j >= 2)
                def _():
                    pltpu.make_async_copy(*cout(j - 2, b2)).wait()

                pltpu.async_copy(*cin(j + 2, b2))

    pltpu.async_copy(*cin(jnp.int32(0), 0))
    pltpu.async_copy(*cin(jnp.int32(1), 1))

    def pair_body(k, carry):
        c = 2 * k
        iwait(c, 0)
        istart(c + 1, 1)
        process_chunk(c, 0)
        for t in range(CSPP // 2):
            cstep(k * CSPP + t)
        iwait(c + 1, 1)

        @pl.when(k < NICH // 2 - 1)
        def _():
            istart(c + 2, 0)

        process_chunk(c + 1, 1)
        for t in range(CSPP // 2, CSPP):
            cstep(k * CSPP + t)
        return carry

    with jax.named_scope("ph_winner"):
        _fori(0, NICH // 2, pair_body, 0)

    with jax.named_scope("ph_copy"):
        for j in range(NCH - 4, NCH):
            pltpu.make_async_copy(*cout(jnp.int32(j), j % 4)).wait()

    def compact(off0, lo_i, hi_i, sub_i):
        def body(v, cnt):
            w = winner[pl.ds(v * L, L)]
            m = (w >= lo_i) & (w < hi_i)
            dst = base + v * L + iota
            plsc.store_compressed(dst_arena.at[pl.ds(off0 + cnt, L)], dst,
                                  mask=m)
            plsc.store_compressed(src_arena.at[pl.ds(off0 + cnt, L)],
                                  w - sub_i, mask=m)
            return cnt + _popcnt(m)

        return _fori(0, RPW // L, body, jnp.int32(0))

    def pad(off0, cnt):
        bd = jnp.take(dst_arena[pl.ds(off0, L)], zid)
        bs = jnp.take(src_arena[pl.ds(off0, L)], zid)
        for t in range(G // L):
            dst_arena[pl.ds(off0 + cnt + t * L, L)] = bd
            src_arena[pl.ds(off0 + cnt + t * L, L)] = bs

    with jax.named_scope("ph_compact"):
        cnt_obs = compact(jnp.int32(0), jnp.int32(0), jnp.int32(A), jnp.int32(0))
        pad(jnp.int32(0), cnt_obs)
        q_obs = (cnt_obs + (G - 1)) // G
        off_new = q_obs * G
        cnt_new = compact(off_new, jnp.int32(A), jnp.int32(TOT),
                          jnp.int32(A))
        pad(off_new, cnt_new)
        q_new = (cnt_new + (G - 1)) // G


    def gs_loop(feat_hbm, off0, q):
        def gstart(j, b):
            pltpu.async_copy(
                feat_hbm.at[src_arena.at[pl.ds(off0 + j * G, G)]],
                stage.at[b], gsem.at[b])

        def gwait(j, b):
            pltpu.make_async_copy(
                feat_hbm.at[src_arena.at[pl.ds(off0 + j * G, G)]],
                stage.at[b], gsem.at[b]).wait()

        def sstart(j, b):
            pltpu.async_copy(
                stage.at[b],
                out_hbm.at[dst_arena.at[pl.ds(off0 + j * G, G)]],
                ssem.at[b])

        def swait(j, b):
            pltpu.make_async_copy(
                stage.at[b],
                out_hbm.at[dst_arena.at[pl.ds(off0 + j * G, G)]],
                ssem.at[b]).wait()

        @pl.when(q > 0)
        def _():
            gstart(jnp.int32(0), jnp.int32(0))

        def body(j, carry):
            b = j % 2
            gwait(j, b)
            sstart(j, b)

            @pl.when(j + 1 < q)
            def _():
                @pl.when(j >= 1)
                def _():
                    swait(j - 1, 1 - b)

                gstart(j + 1, 1 - b)

            return carry

        _fori(0, q, body, 0)

        @pl.when(q > 1)
        def _():
            swait(q - 2, (q - 2) % 2)

        @pl.when(q > 0)
        def _():
            swait(q - 1, (q - 1) % 2)

    with jax.named_scope("ph_gs"):
        gs_loop(obs_hbm, jnp.int32(0), q_obs)
        gs_loop(new_hbm, off_new, q_new)


_sc_scatter = functools.partial(
    pl.kernel,
    out_type=jax.ShapeDtypeStruct((M, D), jnp.float32),
    mesh=plsc.VectorSubcoreMesh(core_axis_name="c", subcore_axis_name="s"),
    scratch_types=[
        pltpu.VMEM((RPW,), jnp.int32),
        pltpu.VMEM((2, ICH), jnp.int32),
        pltpu.VMEM((ARENA,), jnp.int32),
        pltpu.VMEM((ARENA,), jnp.int32),
        pltpu.VMEM((4, G, D), jnp.float32),
        pltpu.SemaphoreType.DMA((2,)),
        pltpu.SemaphoreType.DMA((4,)),
        pltpu.SemaphoreType.DMA((4,)),
    ],
    compiler_params=pltpu.CompilerParams(needs_layout_passes=False),
)(_sc_body)


def _ids_body(act_ref, out_ref):
    act = act_ref[...]
    mx = jnp.max(act)
    out_ref[0:A // D, :] = act
    r = lax.broadcasted_iota(jnp.int32, (N // D, D), 0)
    c = lax.broadcasted_iota(jnp.int32, (N // D, D), 1)
    out_ref[A // D:(A + N) // D, :] = mx + 1 + r * D + c


_ids_kernel = pl.pallas_call(
    _ids_body,
    out_shape=jax.ShapeDtypeStruct(((A + N) // D, D), jnp.int32),
)


def kernel(mem, obs_feat, new_feat, obs_slots, new_slots, active_ids,
           active_det_idx):
    slots = jnp.concatenate([obs_slots, new_slots]).astype(jnp.int32)
    new_mem = _sc_scatter(mem, obs_feat, new_feat, slots)
    act2d = active_ids.astype(jnp.int32).reshape(A // D, D)
    ids = _ids_kernel(act2d).reshape(-1).astype(active_ids.dtype)
    return (new_mem, ids)


# speedup vs baseline: 2.6140x; 1.2446x over previous
import functools

import jax
import jax.numpy as jnp
from jax import lax
from jax.experimental import pallas as pl
from jax.experimental.pallas import tpu as pltpu
from jax.experimental.pallas import tpu_sc as plsc

M = 524288
D = 128
A = 131072
N = 65536
TOT = A + N

NC = 2
NS = 16
NW = NC * NS
L = 16

RPW = M // NW
ICH = 4096
NICH = TOT // ICH
G = 64
ARENA = RPW + 2 * G + L


def _fori(lo, hi, body, init, unroll=None):
    if unroll is None or unroll == 1:
        return lax.fori_loop(jnp.int32(lo), jnp.int32(hi), body, init)
    assert lo == 0 and isinstance(hi, int) and hi % unroll == 0

    def blk(k, carry):
        for u in range(unroll):
            carry = body(k * unroll + u, carry)
        return carry

    return lax.fori_loop(jnp.int32(0), jnp.int32(hi // unroll), blk, init)


def _popcnt(m):
    return jnp.max(plsc.all_reduce_population_count(m))


def _sc_body(mem_hbm, obs_hbm, new_hbm, slots_hbm, out_hbm,
             winner, idxbuf, dst_arena, src_arena, stage,
             isem, gsem, ssem):
    wid = lax.axis_index("s") * NC + lax.axis_index("c")
    base = wid * RPW
    iota = lax.iota(jnp.int32, L)
    zid = jnp.zeros((L,), jnp.int32)
    neg1 = jnp.full((L,), -1, jnp.int32)

    def init_body(k, carry):
        winner[pl.ds(k * L, L)] = neg1
        return carry

    with jax.named_scope("ph_init"):
        _fori(0, RPW // L, init_body, 0, unroll=8)

    def istart(c, b):
        bi = jnp.int32(b)
        return pltpu.async_copy(slots_hbm.at[pl.ds(c * ICH, ICH)],
                                idxbuf.at[bi], isem.at[bi])

    def iwait(c, b):
        bi = jnp.int32(b)
        pltpu.make_async_copy(slots_hbm.at[pl.ds(c * ICH, ICH)],
                              idxbuf.at[bi], isem.at[bi]).wait()

    istart(jnp.int32(0), 0)

    def process_chunk(c, b):
        c0 = c * ICH

        def vec2_blind(k, accv):
            v0 = 2 * k
            sA = idxbuf[jnp.int32(b), pl.ds(v0 * L, L)]
            sB = idxbuf[jnp.int32(b), pl.ds(v0 * L + L, L)]
            localA = sA - base
            localB = sB - base
            inrA = plsc.bitcast(localA, jnp.uint32) < jnp.uint32(RPW)
            inrB = plsc.bitcast(localB, jnp.uint32) < jnp.uint32(RPW)
            lcA = localA & (RPW - 1)
            lcB = localB & (RPW - 1)
            ivA = c0 + v0 * L + iota
            ivB = ivA + L
            plsc.store_scatter(winner, [lcA], ivA, mask=inrA)
            plsc.store_scatter(winner, [lcB], ivB, mask=inrB)
            curA = plsc.load_gather(winner, [lcA], mask=inrA)
            curB = plsc.load_gather(winner, [lcB], mask=inrB)
            lost = (inrA & (curA < ivA)) | (inrB & (curB < ivB))
            return accv | lost.astype(jnp.int32)

        def vec_rmw(v, accv):
            s = idxbuf[jnp.int32(b), pl.ds(v * L, L)]
            local = s - base
            inr = plsc.bitcast(local, jnp.uint32) < jnp.uint32(RPW)
            localc = local & (RPW - 1)
            ivec = c0 + v * L + iota
            cur = plsc.load_gather(winner, [localc], mask=inr)
            upd = inr & (ivec > cur)
            plsc.store_scatter(winner, [localc], ivec, mask=upd)
            cur2 = plsc.load_gather(winner, [localc], mask=upd)
            lost = upd & (cur2 < ivec)
            return accv | lost.astype(jnp.int32)

        def run(vec_body, n, unroll):
            accv = _fori(0, n, vec_body, jnp.zeros((L,), jnp.int32),
                         unroll=unroll)
            return jnp.max(accv)

        lax.while_loop(lambda t: t > 0,
                       lambda _: run(vec_rmw, ICH // L, 8),
                       run(vec2_blind, ICH // L // 2, 4))

    NCH = RPW // G
    CSPP = (NCH + NICH // 2 - 1) // (NICH // 2)

    def cin(j, b):
        return (mem_hbm.at[pl.ds(base + j * G, G)], stage.at[jnp.int32(b)],
                gsem.at[jnp.int32(b)])

    def cout(j, b):
        return (stage.at[jnp.int32(b)], out_hbm.at[pl.ds(base + j * G, G)],
                ssem.at[jnp.int32(b)])

    def cstep(j):
        @pl.when(j < NCH)
        def _():
            b = j % 8
            pltpu.make_async_copy(*cin(j, b)).wait()
            pltpu.async_copy(*cout(j, b))

            @pl.when(j + 4 < NCH)
            def _():
                b4 = (j + 4) % 8

                @pl.when(j >= 4)
                def _():
                    pltpu.make_async_copy(*cout(j - 4, b4)).wait()

                pltpu.async_copy(*cin(j + 4, b4))

    for jp in range(4):
        pltpu.async_copy(*cin(jnp.int32(jp), jp))

    def pair_body(k, carry):
        c = 2 * k
        iwait(c, 0)
        istart(c + 1, 1)
        process_chunk(c, 0)
        for t in range(CSPP // 2):
            cstep(k * CSPP + t)
        iwait(c + 1, 1)

        @pl.when(k < NICH // 2 - 1)
        def _():
            istart(c + 2, 0)

        process_chunk(c + 1, 1)
        for t in range(CSPP // 2, CSPP):
            cstep(k * CSPP + t)
        return carry

    with jax.named_scope("ph_winner"):
        _fori(0, NICH // 2, pair_body, 0)

    with jax.named_scope("ph_copy"):
        for j in range(NCH - 8, NCH):
            pltpu.make_async_copy(*cout(jnp.int32(j), j % 8)).wait()

    def compact(off0, lo_i, hi_i, sub_i):
        def body(v, cnt):
            w = winner[pl.ds(v * L, L)]
            m = (w >= lo_i) & (w < hi_i)
            dst = base + v * L + iota
            plsc.store_compressed(dst_arena.at[pl.ds(off0 + cnt, L)], dst,
                                  mask=m)
            plsc.store_compressed(src_arena.at[pl.ds(off0 + cnt, L)],
                                  w - sub_i, mask=m)
            return cnt + _popcnt(m)

        return _fori(0, RPW // L, body, jnp.int32(0))

    def pad(off0, cnt):
        bd = jnp.take(dst_arena[pl.ds(off0, L)], zid)
        bs = jnp.take(src_arena[pl.ds(off0, L)], zid)
        for t in range(G // L):
            dst_arena[pl.ds(off0 + cnt + t * L, L)] = bd
            src_arena[pl.ds(off0 + cnt + t * L, L)] = bs

    with jax.named_scope("ph_compact"):
        cnt_obs = compact(jnp.int32(0), jnp.int32(0), jnp.int32(A), jnp.int32(0))
        pad(jnp.int32(0), cnt_obs)
        q_obs = (cnt_obs + (G - 1)) // G
        off_new = q_obs * G
        cnt_new = compact(off_new, jnp.int32(A), jnp.int32(TOT),
                          jnp.int32(A))
        pad(off_new, cnt_new)
        q_new = (cnt_new + (G - 1)) // G


    def gs_loop(feat_hbm, off0, q):
        def gstart(j, b):
            pltpu.async_copy(
                feat_hbm.at[src_arena.at[pl.ds(off0 + j * G, G)]],
                stage.at[b], gsem.at[b])

        def gwait(j, b):
            pltpu.make_async_copy(
                feat_hbm.at[src_arena.at[pl.ds(off0 + j * G, G)]],
                stage.at[b], gsem.at[b]).wait()

        def sstart(j, b):
            pltpu.async_copy(
                stage.at[b],
                out_hbm.at[dst_arena.at[pl.ds(off0 + j * G, G)]],
                ssem.at[b])

        def swait(j, b):
            pltpu.make_async_copy(
                stage.at[b],
                out_hbm.at[dst_arena.at[pl.ds(off0 + j * G, G)]],
                ssem.at[b]).wait()

        @pl.when(q > 0)
        def _():
            gstart(jnp.int32(0), jnp.int32(0))

        @pl.when(q > 1)
        def _():
            gstart(jnp.int32(1), jnp.int32(1))

        def body(j, carry):
            b = j % 4
            gwait(j, b)
            sstart(j, b)

            @pl.when(j + 2 < q)
            def _():
                b2 = (j + 2) % 4

                @pl.when(j >= 2)
                def _():
                    swait(j - 2, b2)

                gstart(j + 2, b2)

            return carry

        _fori(0, q, body, 0)

        for t in range(4, 0, -1):
            @pl.when(q > t - 1)
            def _(t=t):
                swait(q - t, (q - t) % 4)

    with jax.named_scope("ph_gs"):
        gs_loop(obs_hbm, jnp.int32(0), q_obs)
        gs_loop(new_hbm, off_new, q_new)


_sc_scatter = functools.partial(
    pl.kernel,
    out_type=jax.ShapeDtypeStruct((M, D), jnp.float32),
    mesh=plsc.VectorSubcoreMesh(core_axis_name="c", subcore_axis_name="s"),
    scratch_types=[
        pltpu.VMEM((RPW,), jnp.int32),
        pltpu.VMEM((2, ICH), jnp.int32),
        pltpu.VMEM((ARENA,), jnp.int32),
        pltpu.VMEM((ARENA,), jnp.int32),
        pltpu.VMEM((8, G, D), jnp.float32),
        pltpu.SemaphoreType.DMA((2,)),
        pltpu.SemaphoreType.DMA((8,)),
        pltpu.SemaphoreType.DMA((8,)),
    ],
    compiler_params=pltpu.CompilerParams(needs_layout_passes=False),
)(_sc_body)


def _ids_body(act_ref, out_ref):
    act = act_ref[...]
    mx = jnp.max(act)
    out_ref[0:A // D, :] = act
    r = lax.broadcasted_iota(jnp.int32, (N // D, D), 0)
    c = lax.broadcasted_iota(jnp.int32, (N // D, D), 1)
    out_ref[A // D:(A + N) // D, :] = mx + 1 + r * D + c


_ids_kernel = pl.pallas_call(
    _ids_body,
    out_shape=jax.ShapeDtypeStruct(((A + N) // D, D), jnp.int32),
)


def kernel(mem, obs_feat, new_feat, obs_slots, new_slots, active_ids,
           active_det_idx):
    slots = jnp.concatenate([obs_slots, new_slots]).astype(jnp.int32)
    new_mem = _sc_scatter(mem, obs_feat, new_feat, slots)
    act2d = active_ids.astype(jnp.int32).reshape(A // D, D)
    ids = _ids_kernel(act2d).reshape(-1).astype(active_ids.dtype)
    return (new_mem, ids)


# speedup vs baseline: 2.7204x; 1.0407x over previous
import functools

import jax
import jax.numpy as jnp
from jax import lax
from jax.experimental import pallas as pl
from jax.experimental.pallas import tpu as pltpu
from jax.experimental.pallas import tpu_sc as plsc

M = 524288
D = 128
A = 131072
N = 65536
TOT = A + N

NC = 2
NS = 16
NW = NC * NS
L = 16

RPW = M // NW
ICH = 4096
NICH = TOT // ICH
G = 64
ARENA = RPW + 2 * G + L


def _fori(lo, hi, body, init, unroll=None):
    if unroll is None or unroll == 1:
        return lax.fori_loop(jnp.int32(lo), jnp.int32(hi), body, init)
    assert lo == 0 and isinstance(hi, int) and hi % unroll == 0

    def blk(k, carry):
        for u in range(unroll):
            carry = body(k * unroll + u, carry)
        return carry

    return lax.fori_loop(jnp.int32(0), jnp.int32(hi // unroll), blk, init)


def _popcnt(m):
    return jnp.max(plsc.all_reduce_population_count(m))


def _sc_body(mem_hbm, obs_hbm, new_hbm, slots_hbm, out_hbm,
             winner, idxbuf, dst_arena, src_arena, stage,
             isem, gsem, ssem):
    wid = lax.axis_index("s") * NC + lax.axis_index("c")
    base = wid * RPW
    iota = lax.iota(jnp.int32, L)
    zid = jnp.zeros((L,), jnp.int32)
    neg1 = jnp.full((L,), -1, jnp.int32)

    def init_body(k, carry):
        winner[pl.ds(k * L, L)] = neg1
        return carry

    with jax.named_scope("ph_init"):
        _fori(0, RPW // L, init_body, 0, unroll=8)

    def istart(c, b):
        bi = jnp.int32(b)
        return pltpu.async_copy(slots_hbm.at[pl.ds(c * ICH, ICH)],
                                idxbuf.at[bi], isem.at[bi])

    def iwait(c, b):
        bi = jnp.int32(b)
        pltpu.make_async_copy(slots_hbm.at[pl.ds(c * ICH, ICH)],
                              idxbuf.at[bi], isem.at[bi]).wait()

    istart(jnp.int32(0), 0)

    def process_chunk(c, b):
        c0 = c * ICH

        def vec2_blind(k, accv):
            v0 = 2 * k
            sA = idxbuf[jnp.int32(b), pl.ds(v0 * L, L)]
            sB = idxbuf[jnp.int32(b), pl.ds(v0 * L + L, L)]
            localA = sA - base
            localB = sB - base
            inrA = plsc.bitcast(localA, jnp.uint32) < jnp.uint32(RPW)
            inrB = plsc.bitcast(localB, jnp.uint32) < jnp.uint32(RPW)
            lcA = localA & (RPW - 1)
            lcB = localB & (RPW - 1)
            ivA = c0 + v0 * L + iota
            ivB = ivA + L
            plsc.store_scatter(winner, [lcA], ivA, mask=inrA)
            plsc.store_scatter(winner, [lcB], ivB, mask=inrB)
            curA = plsc.load_gather(winner, [lcA], mask=inrA)
            curB = plsc.load_gather(winner, [lcB], mask=inrB)
            lost = (inrA & (curA < ivA)) | (inrB & (curB < ivB))
            return accv | lost.astype(jnp.int32)

        def vec_rmw(v, accv):
            s = idxbuf[jnp.int32(b), pl.ds(v * L, L)]
            local = s - base
            inr = plsc.bitcast(local, jnp.uint32) < jnp.uint32(RPW)
            localc = local & (RPW - 1)
            ivec = c0 + v * L + iota
            cur = plsc.load_gather(winner, [localc], mask=inr)
            upd = inr & (ivec > cur)
            plsc.store_scatter(winner, [localc], ivec, mask=upd)
            cur2 = plsc.load_gather(winner, [localc], mask=upd)
            lost = upd & (cur2 < ivec)
            return accv | lost.astype(jnp.int32)

        def run(vec_body, n, unroll):
            accv = _fori(0, n, vec_body, jnp.zeros((L,), jnp.int32),
                         unroll=unroll)
            return jnp.max(accv)

        lax.while_loop(lambda t: t > 0,
                       lambda _: run(vec_rmw, ICH // L, 8),
                       run(vec2_blind, ICH // L // 2, 4))

    NCH = RPW // G
    CSPP = (NCH + NICH // 2 - 1) // (NICH // 2)

    def cin(j, b):
        return (mem_hbm.at[pl.ds(base + j * G, G)], stage.at[jnp.int32(b)],
                gsem.at[jnp.int32(b)])

    def cout(j, b):
        return (stage.at[jnp.int32(b)], out_hbm.at[pl.ds(base + j * G, G)],
                ssem.at[jnp.int32(b)])

    def cstep(j):
        @pl.when(j < NCH)
        def _():
            b = j % 8
            pltpu.make_async_copy(*cin(j, b)).wait()
            pltpu.async_copy(*cout(j, b))

            @pl.when(j + 4 < NCH)
            def _():
                b4 = (j + 4) % 8

                @pl.when(j >= 4)
                def _():
                    pltpu.make_async_copy(*cout(j - 4, b4)).wait()

                pltpu.async_copy(*cin(j + 4, b4))

    for jp in range(4):
        pltpu.async_copy(*cin(jnp.int32(jp), jp))

    def pair_body(k, carry):
        c = 2 * k
        iwait(c, 0)
        istart(c + 1, 1)
        process_chunk(c, 0)
        for t in range(CSPP // 2):
            cstep(k * CSPP + t)
        iwait(c + 1, 1)

        @pl.when(k < NICH // 2 - 1)
        def _():
            istart(c + 2, 0)

        process_chunk(c + 1, 1)
        for t in range(CSPP // 2, CSPP):
            cstep(k * CSPP + t)
        return carry

    with jax.named_scope("ph_winner"):
        _fori(0, NICH // 2, pair_body, 0)


    def compact(off0, lo_i, hi_i, sub_i):
        def body(v, cnt):
            w = winner[pl.ds(v * L, L)]
            m = (w >= lo_i) & (w < hi_i)
            dst = base + v * L + iota
            plsc.store_compressed(dst_arena.at[pl.ds(off0 + cnt, L)], dst,
                                  mask=m)
            plsc.store_compressed(src_arena.at[pl.ds(off0 + cnt, L)],
                                  w - sub_i, mask=m)
            return cnt + _popcnt(m)

        return _fori(0, RPW // L, body, jnp.int32(0))

    def pad(off0, cnt):
        bd = jnp.take(dst_arena[pl.ds(off0, L)], zid)
        bs = jnp.take(src_arena[pl.ds(off0, L)], zid)
        for t in range(G // L):
            dst_arena[pl.ds(off0 + cnt + t * L, L)] = bd
            src_arena[pl.ds(off0 + cnt + t * L, L)] = bs

    with jax.named_scope("ph_compact"):
        cnt_obs = compact(jnp.int32(0), jnp.int32(0), jnp.int32(A), jnp.int32(0))
        pad(jnp.int32(0), cnt_obs)
        q_obs = (cnt_obs + (G - 1)) // G
        off_new = q_obs * G
        cnt_new = compact(off_new, jnp.int32(A), jnp.int32(TOT),
                          jnp.int32(A))
        pad(off_new, cnt_new)
        q_new = (cnt_new + (G - 1)) // G


    with jax.named_scope("ph_copy"):
        for j in range(NCH - 8, NCH):
            pltpu.make_async_copy(*cout(jnp.int32(j), j % 8)).wait()

    def gs_loop(feat_hbm, off0, q):
        def gstart(j, b):
            pltpu.async_copy(
                feat_hbm.at[src_arena.at[pl.ds(off0 + j * G, G)]],
                stage.at[b], gsem.at[b])

        def gwait(j, b):
            pltpu.make_async_copy(
                feat_hbm.at[src_arena.at[pl.ds(off0 + j * G, G)]],
                stage.at[b], gsem.at[b]).wait()

        def sstart(j, b):
            pltpu.async_copy(
                stage.at[b],
                out_hbm.at[dst_arena.at[pl.ds(off0 + j * G, G)]],
                ssem.at[b])

        def swait(j, b):
            pltpu.make_async_copy(
                stage.at[b],
                out_hbm.at[dst_arena.at[pl.ds(off0 + j * G, G)]],
                ssem.at[b]).wait()

        for jp in range(4):
            @pl.when(q > jp)
            def _(jp=jp):
                gstart(jnp.int32(jp), jnp.int32(jp))

        def body(j, carry):
            b = j % 8
            gwait(j, b)
            sstart(j, b)

            @pl.when(j + 4 < q)
            def _():
                b4 = (j + 4) % 8

                @pl.when(j >= 4)
                def _():
                    swait(j - 4, b4)

                gstart(j + 4, b4)

            return carry

        _fori(0, q, body, 0)

        for t in range(8, 0, -1):
            @pl.when(q > t - 1)
            def _(t=t):
                swait(q - t, (q - t) % 8)

    with jax.named_scope("ph_gs"):
        gs_loop(obs_hbm, jnp.int32(0), q_obs)
        gs_loop(new_hbm, off_new, q_new)


_sc_scatter = functools.partial(
    pl.kernel,
    out_type=jax.ShapeDtypeStruct((M, D), jnp.float32),
    mesh=plsc.VectorSubcoreMesh(core_axis_name="c", subcore_axis_name="s"),
    scratch_types=[
        pltpu.VMEM((RPW,), jnp.int32),
        pltpu.VMEM((2, ICH), jnp.int32),
        pltpu.VMEM((ARENA,), jnp.int32),
        pltpu.VMEM((ARENA,), jnp.int32),
        pltpu.VMEM((8, G, D), jnp.float32),
        pltpu.SemaphoreType.DMA((2,)),
        pltpu.SemaphoreType.DMA((8,)),
        pltpu.SemaphoreType.DMA((8,)),
    ],
    compiler_params=pltpu.CompilerParams(needs_layout_passes=False),
)(_sc_body)


def _ids_body(act_ref, out_ref):
    act = act_ref[...]
    mx = jnp.max(act)
    out_ref[0:A // D, :] = act
    r = lax.broadcasted_iota(jnp.int32, (N // D, D), 0)
    c = lax.broadcasted_iota(jnp.int32, (N // D, D), 1)
    out_ref[A // D:(A + N) // D, :] = mx + 1 + r * D + c


_ids_kernel = pl.pallas_call(
    _ids_body,
    out_shape=jax.ShapeDtypeStruct(((A + N) // D, D), jnp.int32),
)


def kernel(mem, obs_feat, new_feat, obs_slots, new_slots, active_ids,
           active_det_idx):
    slots = jnp.concatenate([obs_slots, new_slots]).astype(jnp.int32)
    new_mem = _sc_scatter(mem, obs_feat, new_feat, slots)
    act2d = active_ids.astype(jnp.int32).reshape(A // D, D)
    ids = _ids_kernel(act2d).reshape(-1).astype(active_ids.dtype)
    return (new_mem, ids)
